# Initial kernel scaffold; baseline (speedup 1.0000x reference)
import jax
import jax.numpy as jnp
from jax.experimental import pallas as pl


def kernel(x, Wr, W1, W2):
    raise NotImplementedError("write your pallas kernel here")


# speedup vs baseline: 1.0118x; 1.0118x over previous
import functools

import jax
import jax.numpy as jnp
from jax.experimental import pallas as pl
from jax.experimental.pallas import tpu as pltpu

D_MODEL = 1024
D_FF = 2048
N_EXP = 8
TOK_BLK = 512


def _moe_body(x_ref, wr_ref, w1_ref, w2_ref, out_ref, acc_ref):
    e = pl.program_id(0)
    tb = pl.program_id(1)

    xb = x_ref[...]

    logits = jax.lax.dot_general(
        xb, wr_ref[...], (((1,), (1,)), ((), ())),
        preferred_element_type=jnp.float32)
    p = jax.nn.softmax(logits, axis=-1)
    i1 = jnp.argmax(p, axis=-1)
    v1 = jnp.max(p, axis=-1)
    lane = jax.lax.broadcasted_iota(jnp.int32, p.shape, 1)
    p_masked = jnp.where(lane == i1[:, None], -1.0, p)
    i2 = jnp.argmax(p_masked, axis=-1)
    v2 = jnp.max(p_masked, axis=-1)
    denom = v1 + v2
    w_e = (jnp.where(i1 == e, v1, 0.0) + jnp.where(i2 == e, v2, 0.0)) / denom

    xbb = xb.astype(jnp.bfloat16)
    h = jax.lax.dot_general(
        xbb, w1_ref[0], (((1,), (1,)), ((), ())),
        preferred_element_type=jnp.float32)
    h = (h * jax.nn.sigmoid(h)).astype(jnp.bfloat16)
    y = jax.lax.dot_general(
        h, w2_ref[0], (((1,), (1,)), ((), ())),
        preferred_element_type=jnp.float32)

    contrib = w_e[:, None] * y
    sl = pl.ds(tb * TOK_BLK, TOK_BLK)

    @pl.when(e == 0)
    def _init():
        acc_ref[sl, :] = contrib

    @pl.when(e > 0)
    def _accum():
        acc_ref[sl, :] += contrib

    @pl.when(e == N_EXP - 1)
    def _emit():
        out_ref[...] = acc_ref[sl, :]


@jax.jit
def kernel(x, Wr, W1, W2):
    b, s, d = x.shape
    n = b * s
    x2 = x.reshape(n, d)
    w1b = W1.astype(jnp.bfloat16)
    w2b = W2.astype(jnp.bfloat16)
    n_tb = n // TOK_BLK

    out = pl.pallas_call(
        _moe_body,
        grid=(N_EXP, n_tb),
        in_specs=[
            pl.BlockSpec((TOK_BLK, D_MODEL), lambda e, tb: (tb, 0)),
            pl.BlockSpec((N_EXP, D_MODEL), lambda e, tb: (0, 0)),
            pl.BlockSpec((1, D_FF, D_MODEL), lambda e, tb: (e, 0, 0)),
            pl.BlockSpec((1, D_MODEL, D_FF), lambda e, tb: (e, 0, 0)),
        ],
        out_specs=pl.BlockSpec((TOK_BLK, D_MODEL), lambda e, tb: (tb, 0)),
        out_shape=jax.ShapeDtypeStruct((n, D_MODEL), jnp.float32),
        scratch_shapes=[pltpu.VMEM((n, D_MODEL), jnp.float32)],
    )(x2, Wr, w1b, w2b)
    return out.reshape(b, s, d)


# speedup vs baseline: 1.4114x; 1.3950x over previous
import functools

import jax
import jax.numpy as jnp
from jax import lax
from jax.experimental import pallas as pl
from jax.experimental.pallas import tpu as pltpu
from jax.experimental.pallas import tpu_sc as plsc

D_MODEL = 1024
D_FF = 2048
N_EXP = 8
N_TOK = 4096
N_PAIR = 2 * N_TOK
TD = 256
NB = N_PAIR // TD + N_EXP
NPAD = NB * TD
NW = 32
TPW = N_TOK // NW
CH = 32
WGL = 128

_HI = jax.lax.Precision.HIGHEST


def _router_body(x_ref, wr_ref, i1_ref, i2_ref, wa_ref, wb_ref):
    xb = x_ref[...]
    logits = lax.dot_general(xb, wr_ref[...], (((1,), (1,)), ((), ())),
                             preferred_element_type=jnp.float32)
    p = jax.nn.softmax(logits, axis=-1)
    i1 = jnp.argmax(p, axis=-1)
    v1 = jnp.max(p, axis=-1)
    lane = lax.broadcasted_iota(jnp.int32, p.shape, 1)
    pm = jnp.where(lane == i1[:, None], -1.0, p)
    i2 = jnp.argmax(pm, axis=-1)
    v2 = jnp.max(pm, axis=-1)
    denom = v1 + v2
    i1_ref[...] = i1.astype(jnp.int32)
    i2_ref[...] = i2.astype(jnp.int32)
    wa_ref[...] = v1 / denom
    wb_ref[...] = v2 / denom


def _book_body(pe_ref, pos_ref, beo_ref):
    pe = pe_ref[...]
    r128 = lax.broadcasted_iota(jnp.int32, (128, 128), 0)
    c128 = lax.broadcasted_iota(jnp.int32, (128, 128), 1)
    U = (r128 <= c128).astype(jnp.float32)
    r64 = lax.broadcasted_iota(jnp.int32, (64, 64), 0)
    c64 = lax.broadcasted_iota(jnp.int32, (64, 64), 1)
    L = (c64 < r64).astype(jnp.float32)

    pos_acc = jnp.zeros((64, 128), jnp.float32)
    be_acc = jnp.zeros((8, 128), jnp.int32)
    bidx = lax.broadcasted_iota(jnp.int32, (8, 128), 1)
    offs = jnp.int32(0)
    for e in range(N_EXP):
        m = (pe == e).astype(jnp.float32)
        cs = lax.dot_general(m, U, (((1,), (0,)), ((), ())),
                             precision=_HI, preferred_element_type=jnp.float32)
        rt = cs[:, 127:128]
        O = lax.dot_general(L, rt, (((1,), (0,)), ((), ())),
                            precision=_HI, preferred_element_type=jnp.float32)
        cnt = jnp.sum(rt).astype(jnp.int32)
        offs_f = offs.astype(jnp.float32)
        pos_acc = pos_acc + m * (offs_f + O + cs - 1.0)
        be_acc = be_acc + (bidx >= offs // TD).astype(jnp.int32)
        pc = ((cnt + TD - 1) // TD) * TD
        offs = offs + pc
    pos_ref[...] = pos_acc.astype(jnp.int32)
    beo_ref[...] = be_acc - 1


def _dispatch_body(x_hbm, pose_hbm, poso_hbm, w0_hbm, w1_hbm,
                   xg_hbm, wg_hbm,
                   pose_v, poso_v, w0_v, w1_v, rows_v, wg0_v, wg1_v, sem):
    wid = lax.axis_index("s") * 2 + lax.axis_index("c")
    pltpu.sync_copy(pose_hbm.at[wid], pose_v)
    pltpu.sync_copy(poso_hbm.at[wid], poso_v)
    pltpu.sync_copy(w0_hbm.at[pl.ds(wid * TPW, TPW)], w0_v)
    pltpu.sync_copy(w1_hbm.at[pl.ds(wid * TPW, TPW)], w1_v)
    for c in range(TPW // CH):
        base = wid * TPW + c * CH
        pltpu.sync_copy(x_hbm.at[pl.ds(base, CH)], rows_v)
        for i in range(CH):
            v0 = w0_v[pl.ds(c * CH + (i // 16) * 16, 16)]
            v1 = w1_v[pl.ds(c * CH + (i // 16) * 16, 16)]
            f0 = jnp.full((16,), v0[i % 16], jnp.float32)
            f1 = jnp.full((16,), v1[i % 16], jnp.float32)
            for j in range(WGL // 16):
                wg0_v[i, pl.ds(j * 16, 16)] = f0
                wg1_v[i, pl.ds(j * 16, 16)] = f1
        pltpu.async_copy(rows_v, xg_hbm.at[pose_v.at[c]], sem).wait()
        pltpu.async_copy(rows_v, xg_hbm.at[poso_v.at[c]], sem).wait()
        pltpu.async_copy(wg0_v, wg_hbm.at[pose_v.at[c]], sem).wait()
        pltpu.async_copy(wg1_v, wg_hbm.at[poso_v.at[c]], sem).wait()


def _ffn_body(be_ref, xg_ref, w1_ref, w2_ref, wg_ref, yg_ref):
    xb = xg_ref[...].astype(jnp.bfloat16)
    h = lax.dot_general(xb, w1_ref[0], (((1,), (1,)), ((), ())),
                        preferred_element_type=jnp.float32)
    h = (h * jax.nn.sigmoid(h)).astype(jnp.bfloat16)
    y = lax.dot_general(h, w2_ref[0], (((1,), (1,)), ((), ())),
                        preferred_element_type=jnp.float32)
    yg_ref[...] = y * wg_ref[...][:, 0:1]


def _combine_body(yg_hbm, pose_hbm, poso_hbm, out_hbm,
                  pose_v, poso_v, ye_v, yo_v, o_v, sem):
    wid = lax.axis_index("s") * 2 + lax.axis_index("c")
    pltpu.sync_copy(pose_hbm.at[wid], pose_v)
    pltpu.sync_copy(poso_hbm.at[wid], poso_v)
    for c in range(TPW // CH):
        pltpu.async_copy(yg_hbm.at[pose_v.at[c]], ye_v, sem).wait()
        pltpu.async_copy(yg_hbm.at[poso_v.at[c]], yo_v, sem).wait()

        def row(i, _):
            for j in range(D_MODEL // 16):
                sl = pl.ds(j * 16, 16)
                o_v[i, sl] = ye_v[i, sl] + yo_v[i, sl]
            return 0

        lax.fori_loop(0, CH, row, 0)
        pltpu.sync_copy(o_v, out_hbm.at[pl.ds(wid * TPW + c * CH, CH)])


@jax.jit
def kernel(x, Wr, W1, W2):
    b, s, d = x.shape
    x2 = x.reshape(N_TOK, d)
    w1b = W1.astype(jnp.bfloat16)
    w2b = W2.astype(jnp.bfloat16)

    i1, i2, wa, wb = pl.pallas_call(
        _router_body,
        out_shape=[
            jax.ShapeDtypeStruct((N_TOK,), jnp.int32),
            jax.ShapeDtypeStruct((N_TOK,), jnp.int32),
            jax.ShapeDtypeStruct((N_TOK,), jnp.float32),
            jax.ShapeDtypeStruct((N_TOK,), jnp.float32),
        ],
    )(x2, Wr)

    pair_e = jnp.stack([i1, i2], axis=1).reshape(64, 128)
    pos, beo = pl.pallas_call(
        _book_body,
        out_shape=[
            jax.ShapeDtypeStruct((64, 128), jnp.int32),
            jax.ShapeDtypeStruct((8, 128), jnp.int32),
        ],
    )(pair_e)

    pos2 = pos.reshape(N_TOK, 2)
    pose = pos2[:, 0].reshape(NW, TPW // CH, CH)
    poso = pos2[:, 1].reshape(NW, TPW // CH, CH)
    be = beo[0, :NB]

    mesh = plsc.VectorSubcoreMesh(core_axis_name="c", subcore_axis_name="s")
    dispatch = functools.partial(
        pl.kernel,
        mesh=mesh,
        out_type=[
            jax.ShapeDtypeStruct((NPAD, D_MODEL), jnp.float32),
            jax.ShapeDtypeStruct((NPAD, WGL), jnp.float32),
        ],
        scratch_types=[
            pltpu.VMEM((TPW // CH, CH), jnp.int32),
            pltpu.VMEM((TPW // CH, CH), jnp.int32),
            pltpu.VMEM((TPW,), jnp.float32),
            pltpu.VMEM((TPW,), jnp.float32),
            pltpu.VMEM((CH, D_MODEL), jnp.float32),
            pltpu.VMEM((CH, WGL), jnp.float32),
            pltpu.VMEM((CH, WGL), jnp.float32),
            pltpu.SemaphoreType.DMA,
        ],
    )(_dispatch_body)
    xg, wg = dispatch(x2, pose, poso, wa, wb)

    grid_spec = pltpu.PrefetchScalarGridSpec(
        num_scalar_prefetch=1,
        grid=(NB,),
        in_specs=[
            pl.BlockSpec((TD, D_MODEL), lambda bb, be_r: (bb, 0)),
            pl.BlockSpec((1, D_FF, D_MODEL), lambda bb, be_r: (be_r[bb], 0, 0)),
            pl.BlockSpec((1, D_MODEL, D_FF), lambda bb, be_r: (be_r[bb], 0, 0)),
            pl.BlockSpec((TD, WGL), lambda bb, be_r: (bb, 0)),
        ],
        out_specs=pl.BlockSpec((TD, D_MODEL), lambda bb, be_r: (bb, 0)),
    )
    yg = pl.pallas_call(
        _ffn_body,
        grid_spec=grid_spec,
        out_shape=jax.ShapeDtypeStruct((NPAD, D_MODEL), jnp.float32),
    )(be, xg, w1b, w2b, wg)

    combine = functools.partial(
        pl.kernel,
        mesh=mesh,
        out_type=jax.ShapeDtypeStruct((N_TOK, D_MODEL), jnp.float32),
        scratch_types=[
            pltpu.VMEM((TPW // CH, CH), jnp.int32),
            pltpu.VMEM((TPW // CH, CH), jnp.int32),
            pltpu.VMEM((CH, D_MODEL), jnp.float32),
            pltpu.VMEM((CH, D_MODEL), jnp.float32),
            pltpu.VMEM((CH, D_MODEL), jnp.float32),
            pltpu.SemaphoreType.DMA,
        ],
    )(_combine_body)
    out = combine(yg, pose, poso)
    return out.reshape(b, s, d)
